# baseline (speedup 1.0000x reference)
import jax
import jax.numpy as jnp
from jax.experimental import pallas as pl
from jax.experimental.pallas import tpu as pltpu

_BS = 64
_N = 3000
_NPAD = 3072
_LC = 1024
_N_LC = _NPAD // _LC
_IC = 16
_SYM = (12, 15, 18, 19, 20)
_PADVAL = 1e15


def _loss_kernel(params_ref, mpT_ref, mp_ref, tgt8_ref, out_ref, r2_ref):
    def p(k):
        return params_ref[0, 0, k]

    m = p(12)

    @pl.when(m > 0.5)
    def _sym():
        cx8 = tgt8_ref[0, 0]
        cy8 = tgt8_ref[0, 1]
        cz8 = tgt8_ref[0, 2]
        r2_ref[:, :] = cx8 * cx8 + cy8 * cy8 + cz8 * cz8

        def chunk(ci, carry):
            t0, t1 = carry
            i0 = ci * _IC
            mch = mp_ref[0, pl.ds(i0, _IC), :]
            mx = mch[:, 0:1]
            my = mch[:, 1:2]
            mz = mch[:, 2:3]
            bx = mx * (-2.0 * p(0)) + my * (-2.0 * p(3)) + mz * (-2.0 * p(6)) - 2.0 * p(9)
            by = mx * (-2.0 * p(1)) + my * (-2.0 * p(4)) + mz * (-2.0 * p(7)) - 2.0 * p(10)
            bz = mx * (-2.0 * p(2)) + my * (-2.0 * p(5)) + mz * (-2.0 * p(8)) - 2.0 * p(11)
            acc0 = None
            acc1 = None
            for c in range(_N_LC):
                sl = slice(c * _LC, (c + 1) * _LC)
                cx = tgt8_ref[0, 0, :, sl]
                cy = tgt8_ref[0, 1, :, sl]
                cz = tgt8_ref[0, 2, :, sl]
                cr = r2_ref[:, sl]
                v0 = cr + cx * bx[0:8] + cy * by[0:8] + cz * bz[0:8]
                v1 = cr + cx * bx[8:16] + cy * by[8:16] + cz * bz[8:16]
                acc0 = v0 if acc0 is None else jnp.minimum(acc0, v0)
                acc1 = v1 if acc1 is None else jnp.minimum(acc1, v1)
            m0 = jnp.min(acc0, axis=1, keepdims=True)
            m1 = jnp.min(acc1, axis=1, keepdims=True)
            q0 = 0.25 * (bx[0:8] * bx[0:8] + by[0:8] * by[0:8] + bz[0:8] * bz[0:8])
            q1 = 0.25 * (bx[8:16] * bx[8:16] + by[8:16] * by[8:16] + bz[8:16] * bz[8:16])
            d0 = jnp.sqrt(jnp.maximum(m0 + q0, 0.0))
            d1 = jnp.sqrt(jnp.maximum(m1 + q1, 0.0))
            row = jax.lax.broadcasted_iota(jnp.int32, (8, 1), 0) + i0
            w0 = (row < _N).astype(jnp.float32)
            w1 = ((row + 8) < _N).astype(jnp.float32)
            return (t0 + d0 * w0, t1 + d1 * w1)

        z = jnp.zeros((8, 1), dtype=jnp.float32)
        t0, t1 = jax.lax.fori_loop(0, _NPAD // _IC, chunk, (z, z))
        s = jnp.sum(t0 + t1, axis=0, keepdims=True)
        out_ref[0] = s

    @pl.when(m <= 0.5)
    def _plain():
        mpx = mpT_ref[0, 0:1, :]
        mpy = mpT_ref[0, 1:2, :]
        mpz = mpT_ref[0, 2:3, :]
        tfx = mpx * p(0) + mpy * p(3) + mpz * p(6) + p(9)
        tfy = mpx * p(1) + mpy * p(4) + mpz * p(7) + p(10)
        tfz = mpx * p(2) + mpy * p(5) + mpz * p(8) + p(11)
        dx = tfx - tgt8_ref[0, 0, 0:1, :]
        dy = tfy - tgt8_ref[0, 1, 0:1, :]
        dz = tfz - tgt8_ref[0, 2, 0:1, :]
        d2 = dx * dx + dy * dy + dz * dz
        lane = jax.lax.broadcasted_iota(jnp.int32, (1, _NPAD), 1)
        lvalid = (lane < _N).astype(jnp.float32)
        s = jnp.sum(jnp.sqrt(d2) * lvalid, axis=1, keepdims=True)
        out_ref[0] = s


def kernel(pred_r, pred_t, target, model_points, idx):
    bs, num_p, _ = target.shape

    q = pred_r / jnp.linalg.norm(pred_r, axis=1, keepdims=True)
    w, x, y, z = q[:, 0], q[:, 1], q[:, 2], q[:, 3]
    r00 = 1.0 - 2.0 * (y * y + z * z)
    r01 = 2.0 * (x * y - w * z)
    r02 = 2.0 * (x * z + w * y)
    r10 = 2.0 * (x * y + w * z)
    r11 = 1.0 - 2.0 * (x * x + z * z)
    r12 = 2.0 * (y * z - w * x)
    r20 = 2.0 * (x * z - w * y)
    r21 = 2.0 * (y * z + w * x)
    r22 = 1.0 - 2.0 * (x * x + y * y)
    sym = jnp.asarray(_SYM, dtype=idx.dtype)
    mask = (idx[:, 0][:, None] == sym[None, :]).any(axis=1).astype(jnp.float32)
    zeros = jnp.zeros_like(w)
    params = jnp.stack(
        [r00, r01, r02, r10, r11, r12, r20, r21, r22,
         pred_t[:, 0], pred_t[:, 1], pred_t[:, 2], mask, zeros, zeros, zeros],
        axis=1).reshape(bs, 1, 16)

    pad_n = _NPAD - num_p
    mpT = jnp.pad(jnp.transpose(model_points, (0, 2, 1)),
                  ((0, 0), (0, 0), (0, pad_n)))
    mp_p = jnp.pad(model_points, ((0, 0), (0, pad_n), (0, 0)))
    tgtT = jnp.pad(jnp.transpose(target, (0, 2, 1)),
                   ((0, 0), (0, 0), (0, pad_n)), constant_values=_PADVAL)
    tgt8 = jnp.broadcast_to(tgtT[:, :, None, :], (bs, 3, 8, _NPAD))

    out = pl.pallas_call(
        _loss_kernel,
        grid=(bs,),
        in_specs=[
            pl.BlockSpec((1, 1, 16), lambda b: (b, 0, 0), memory_space=pltpu.SMEM),
            pl.BlockSpec((1, 3, _NPAD), lambda b: (b, 0, 0)),
            pl.BlockSpec((1, _NPAD, 3), lambda b: (b, 0, 0)),
            pl.BlockSpec((1, 3, 8, _NPAD), lambda b: (b, 0, 0, 0)),
        ],
        out_specs=pl.BlockSpec((1, 1, 1), lambda b: (b, 0, 0)),
        out_shape=jax.ShapeDtypeStruct((bs, 1, 1), jnp.float32),
        scratch_shapes=[pltpu.VMEM((8, _NPAD), jnp.float32)],
    )(params, mpT, mp_p, tgt8)

    return out[:, 0, 0] / jnp.float32(num_p)
